# Initial kernel scaffold; baseline (speedup 1.0000x reference)
import jax
import jax.numpy as jnp
from jax.experimental import pallas as pl


def kernel(input, edge_data, weight, bias, Esrc, Etgt):
    raise NotImplementedError("write your pallas kernel here")


# speedup vs baseline: 2.3770x; 2.3770x over previous
import functools

import jax
import jax.numpy as jnp
from jax import lax
from jax.experimental import pallas as pl
from jax.experimental.pallas import tpu as pltpu
from jax.experimental.pallas import tpu_sc as plsc

F32 = jnp.float32

_NC = 2
_NS = 16
_NW = _NC * _NS
_CH = 128


def _support_matmul(x, w):
    n, din = x.shape
    dout = w.shape[1]
    rb = 2000

    def body(x_ref, w_ref, o_ref):
        o_ref[...] = jnp.dot(x_ref[...], w_ref[...],
                             preferred_element_type=F32)

    return pl.pallas_call(
        body,
        grid=(n // rb,),
        in_specs=[pl.BlockSpec((rb, din), lambda i: (i, 0)),
                  pl.BlockSpec((din, dout), lambda i: (0, 0))],
        out_specs=pl.BlockSpec((rb, dout), lambda i: (i, 0)),
        out_shape=jax.ShapeDtypeStruct((n, dout), F32),
    )(x, w)


def _sc_gather(table, idx3):
    nw, nch, ch = idx3.shape
    d = table.shape[1]
    epw = nch * ch
    e_pad = nw * epw
    mesh = plsc.VectorSubcoreMesh(core_axis_name="c", subcore_axis_name="s")

    @functools.partial(
        pl.kernel,
        out_type=jax.ShapeDtypeStruct((e_pad, d), F32),
        mesh=mesh,
        scratch_types=[pltpu.VMEM((nch, ch), jnp.int32),
                       pltpu.VMEM((ch, d), F32),
                       pltpu.SemaphoreType.DMA],
        compiler_params=pltpu.CompilerParams(use_tc_tiling_on_sc=False),
    )
    def k(tbl_hbm, idx_hbm, out_hbm, idx_v, rows_v, sem):
        c = lax.axis_index("c")
        s = lax.axis_index("s")
        wid = s * _NC + c
        base = wid * epw
        pltpu.sync_copy(idx_hbm.at[wid], idx_v)

        def body(j, carry):
            pltpu.async_copy(tbl_hbm.at[idx_v.at[j]], rows_v, sem).wait()
            pltpu.sync_copy(rows_v, out_hbm.at[pl.ds(base + j * ch, ch)])
            return carry

        lax.fori_loop(0, nch, body, 0)

    return k(table, idx3)


def _bmm(ed2, es, t_mat, s_mat, e_real, blk):
    dd = ed2.shape[1]
    e_pad, dout = es.shape
    g = e_pad // blk
    g_real = e_real // blk

    def body(ed_ref, es_ref, t_ref, s_ref, o_ref):
        rep = jnp.dot(es_ref[...], t_ref[...], preferred_element_type=F32)
        o_ref[...] = jnp.dot(ed_ref[...] * rep, s_ref[...],
                             preferred_element_type=F32)

    return pl.pallas_call(
        body,
        grid=(g,),
        in_specs=[pl.BlockSpec((blk, dd), lambda i: (jnp.minimum(i, g_real - 1), 0)),
                  pl.BlockSpec((blk, dout), lambda i: (i, 0)),
                  pl.BlockSpec((dout, dd), lambda i: (0, 0)),
                  pl.BlockSpec((dd, dout), lambda i: (0, 0))],
        out_specs=pl.BlockSpec((blk, dout), lambda i: (i, 0)),
        out_shape=jax.ShapeDtypeStruct((e_pad, dout), F32),
    )(ed2, es, t_mat, s_mat)


def _sc_scatter(msg, idx3, zeros):
    nw, nch, ch = idx3.shape
    d = msg.shape[1]
    acc = zeros.shape[0]
    rpt = acc // _NS
    epw = nch * ch
    mesh = plsc.VectorSubcoreMesh(core_axis_name="c", subcore_axis_name="s")

    @functools.partial(
        pl.kernel,
        out_type=jax.ShapeDtypeStruct((_NC, acc, d), F32),
        mesh=mesh,
        scratch_types=[pltpu.VMEM((nch, ch), jnp.int32),
                       pltpu.VMEM((ch, d), F32),
                       pltpu.VMEM_SHARED((acc, d), F32),
                       pltpu.SemaphoreType.DMA],
        compiler_params=pltpu.CompilerParams(use_tc_tiling_on_sc=False),
    )
    def k(msg_hbm, idx_hbm, z_hbm, out_hbm, idx_v, msg_v, acc_sh, sem):
        c = lax.axis_index("c")
        s = lax.axis_index("s")
        wid = s * _NC + c
        base = wid * epw
        pltpu.sync_copy(idx_hbm.at[wid], idx_v)
        pltpu.sync_copy(z_hbm.at[pl.ds(s * rpt, rpt)],
                        acc_sh.at[pl.ds(s * rpt, rpt)])
        plsc.subcore_barrier()

        def body(j, carry):
            pltpu.sync_copy(msg_hbm.at[pl.ds(base + j * ch, ch)], msg_v)
            pltpu.sync_copy(msg_v, acc_sh.at[idx_v.at[j]], add=True)
            return carry

        lax.fori_loop(0, nch, body, 0)
        plsc.subcore_barrier()
        pltpu.sync_copy(acc_sh.at[pl.ds(s * rpt, rpt)],
                        out_hbm.at[c, pl.ds(s * rpt, rpt)])

    return k(msg, idx3, zeros)


def _combine(parts, bias8, n):
    dout = parts.shape[2]
    rb = 2000

    def body(p_ref, b_ref, o_ref):
        o_ref[...] = p_ref[0] + p_ref[1] + b_ref[0]

    return pl.pallas_call(
        body,
        grid=(n // rb,),
        in_specs=[pl.BlockSpec((2, rb, dout), lambda i: (0, i, 0)),
                  pl.BlockSpec((8, dout), lambda i: (0, 0))],
        out_specs=pl.BlockSpec((rb, dout), lambda i: (i, 0)),
        out_shape=jax.ShapeDtypeStruct((n, dout), F32),
    )(parts, bias8)


def kernel(input, edge_data, weight, bias, Esrc, Etgt):
    n, din = input.shape
    e = Esrc.shape[0]
    dout = weight.shape[1]
    dd = dout * dout

    epw = -(-e // (_NW * _CH)) * _CH
    e_pad = _NW * epw
    nch = epw // _CH
    acc = -(-(n + 1) // _NS) * _NS

    support = _support_matmul(input, weight)

    esrc3 = jnp.pad(Esrc, (0, e_pad - e)).reshape(_NW, nch, _CH)
    etgt3 = jnp.pad(Etgt, (0, e_pad - e),
                    constant_values=n).reshape(_NW, nch, _CH)

    edge_support = _sc_gather(support, esrc3)

    ed2 = edge_data.reshape(e, dd)
    kk = jnp.arange(dd)
    t_mat = (jnp.arange(dout)[:, None] == (kk[None, :] % dout)).astype(F32)
    s_mat = (kk[:, None] // dout == jnp.arange(dout)[None, :]).astype(F32)
    msg = _bmm(ed2, edge_support, t_mat, s_mat, e, 256)

    zeros = jnp.zeros((acc, dout), F32)
    parts = _sc_scatter(msg, etgt3, zeros)

    bias8 = jnp.broadcast_to(bias[None, :], (8, dout))
    return _combine(parts, bias8, n)


# speedup vs baseline: 5.9635x; 2.5089x over previous
import functools

import jax
import jax.numpy as jnp
from jax import lax
from jax.experimental import pallas as pl
from jax.experimental.pallas import tpu as pltpu
from jax.experimental.pallas import tpu_sc as plsc

F32 = jnp.float32

_NC = 2
_NS = 16
_NW = _NC * _NS
_CH = 128


def _support_matmul(x, w):
    n, din = x.shape
    dout = w.shape[1]
    rb = 2000

    def body(x_ref, w_ref, o_ref):
        o_ref[...] = jnp.dot(x_ref[...], w_ref[...],
                             preferred_element_type=F32)

    return pl.pallas_call(
        body,
        grid=(n // rb,),
        in_specs=[pl.BlockSpec((rb, din), lambda i: (i, 0)),
                  pl.BlockSpec((din, dout), lambda i: (0, 0))],
        out_specs=pl.BlockSpec((rb, dout), lambda i: (i, 0)),
        out_shape=jax.ShapeDtypeStruct((n, dout), F32),
    )(x, w)


def _sc_gather(table, idx3):
    nw, nch, ch = idx3.shape
    d = table.shape[1]
    epw = nch * ch
    e_pad = nw * epw
    mesh = plsc.VectorSubcoreMesh(core_axis_name="c", subcore_axis_name="s")

    @functools.partial(
        pl.kernel,
        out_type=jax.ShapeDtypeStruct((e_pad, d), F32),
        mesh=mesh,
        scratch_types=[pltpu.VMEM((nch, ch), jnp.int32),
                       pltpu.VMEM((ch, d), F32),
                       pltpu.SemaphoreType.DMA],
        compiler_params=pltpu.CompilerParams(use_tc_tiling_on_sc=False),
    )
    def k(tbl_hbm, idx_hbm, out_hbm, idx_v, rows_v, sem):
        c = lax.axis_index("c")
        s = lax.axis_index("s")
        wid = s * _NC + c
        base = wid * epw
        pltpu.sync_copy(idx_hbm.at[wid], idx_v)

        def body(j, carry):
            pltpu.async_copy(tbl_hbm.at[idx_v.at[j]], rows_v, sem).wait()
            pltpu.sync_copy(rows_v, out_hbm.at[pl.ds(base + j * ch, ch)])
            return carry

        lax.fori_loop(0, nch, body, 0)

    return k(table, idx3)


def _bmm_t(edt, es, e_real, blk):
    dout = es.shape[1]
    e_pad = es.shape[0]
    g = e_pad // blk
    gmax = -(-e_real // blk) - 1

    def body(ed_ref, es_ref, o_ref):
        est = jnp.transpose(es_ref[...])
        prod = ed_ref[...] * est[None, :, :]
        msgt = jnp.sum(prod, axis=1)
        o_ref[...] = jnp.transpose(msgt)

    return pl.pallas_call(
        body,
        grid=(g,),
        in_specs=[pl.BlockSpec((dout, dout, blk),
                               lambda i: (0, 0, jnp.minimum(i, gmax))),
                  pl.BlockSpec((blk, dout), lambda i: (i, 0))],
        out_specs=pl.BlockSpec((blk, dout), lambda i: (i, 0)),
        out_shape=jax.ShapeDtypeStruct((e_pad, dout), F32),
    )(edt, es)


def _sc_scatter(msg, idx3, zeros):
    nw, nch, ch = idx3.shape
    d = msg.shape[1]
    acc = zeros.shape[0]
    rpt = acc // _NS
    epw = nch * ch
    mesh = plsc.VectorSubcoreMesh(core_axis_name="c", subcore_axis_name="s")

    @functools.partial(
        pl.kernel,
        out_type=jax.ShapeDtypeStruct((_NC, acc, d), F32),
        mesh=mesh,
        scratch_types=[pltpu.VMEM((nch, ch), jnp.int32),
                       pltpu.VMEM((ch, d), F32),
                       pltpu.VMEM_SHARED((acc, d), F32),
                       pltpu.SemaphoreType.DMA],
        compiler_params=pltpu.CompilerParams(use_tc_tiling_on_sc=False),
    )
    def k(msg_hbm, idx_hbm, z_hbm, out_hbm, idx_v, msg_v, acc_sh, sem):
        c = lax.axis_index("c")
        s = lax.axis_index("s")
        wid = s * _NC + c
        base = wid * epw
        pltpu.sync_copy(idx_hbm.at[wid], idx_v)
        pltpu.sync_copy(z_hbm.at[pl.ds(s * rpt, rpt)],
                        acc_sh.at[pl.ds(s * rpt, rpt)])
        plsc.subcore_barrier()

        def body(j, carry):
            pltpu.sync_copy(msg_hbm.at[pl.ds(base + j * ch, ch)], msg_v)
            pltpu.sync_copy(msg_v, acc_sh.at[idx_v.at[j]], add=True)
            return carry

        lax.fori_loop(0, nch, body, 0)
        plsc.subcore_barrier()
        pltpu.sync_copy(acc_sh.at[pl.ds(s * rpt, rpt)],
                        out_hbm.at[c, pl.ds(s * rpt, rpt)])

    return k(msg, idx3, zeros)


def _combine(parts, bias8, n):
    dout = parts.shape[2]
    rb = 2000

    def body(p_ref, b_ref, o_ref):
        o_ref[...] = p_ref[0] + p_ref[1] + b_ref[0]

    return pl.pallas_call(
        body,
        grid=(n // rb,),
        in_specs=[pl.BlockSpec((2, rb, dout), lambda i: (0, i, 0)),
                  pl.BlockSpec((8, dout), lambda i: (0, 0))],
        out_specs=pl.BlockSpec((rb, dout), lambda i: (i, 0)),
        out_shape=jax.ShapeDtypeStruct((n, dout), F32),
    )(parts, bias8)


def kernel(input, edge_data, weight, bias, Esrc, Etgt):
    n, din = input.shape
    e = Esrc.shape[0]
    dout = weight.shape[1]

    epw = -(-e // (_NW * _CH)) * _CH
    e_pad = _NW * epw
    nch = epw // _CH
    acc = -(-(n + 1) // _NS) * _NS

    support = _support_matmul(input, weight)

    esrc3 = jnp.pad(Esrc, (0, e_pad - e)).reshape(_NW, nch, _CH)
    etgt3 = jnp.pad(Etgt, (0, e_pad - e),
                    constant_values=n).reshape(_NW, nch, _CH)

    edge_support = _sc_gather(support, esrc3)

    edt = jnp.transpose(edge_data, (1, 2, 0))
    msg = _bmm_t(edt, edge_support, e, 1024)

    zeros = jnp.zeros((acc, dout), F32)
    parts = _sc_scatter(msg, etgt3, zeros)

    bias8 = jnp.broadcast_to(bias[None, :], (8, dout))
    return _combine(parts, bias8, n)


# speedup vs baseline: 6.8482x; 1.1483x over previous
import functools

import jax
import jax.numpy as jnp
from jax import lax
from jax.experimental import pallas as pl
from jax.experimental.pallas import tpu as pltpu
from jax.experimental.pallas import tpu_sc as plsc

F32 = jnp.float32

_NC = 2
_NS = 16
_NW = _NC * _NS
_CH = 128


def _support_matmul(x, w):
    n, din = x.shape
    dout = w.shape[1]
    rb = 2000

    def body(x_ref, w_ref, o_ref):
        o_ref[...] = jnp.dot(x_ref[...], w_ref[...],
                             preferred_element_type=F32)

    return pl.pallas_call(
        body,
        grid=(n // rb,),
        in_specs=[pl.BlockSpec((rb, din), lambda i: (i, 0)),
                  pl.BlockSpec((din, dout), lambda i: (0, 0))],
        out_specs=pl.BlockSpec((rb, dout), lambda i: (i, 0)),
        out_shape=jax.ShapeDtypeStruct((n, dout), F32),
    )(x, w)


def _sc_gather(table, idx3):
    nw, nch, ch = idx3.shape
    d = table.shape[1]
    epw = nch * ch
    e_pad = nw * epw
    mesh = plsc.VectorSubcoreMesh(core_axis_name="c", subcore_axis_name="s")

    nb = 8
    assert nch % nb == 0

    @functools.partial(
        pl.kernel,
        out_type=jax.ShapeDtypeStruct((e_pad, d), F32),
        mesh=mesh,
        scratch_types=[pltpu.VMEM((nch, ch), jnp.int32),
                       [pltpu.VMEM((ch, d), F32)] * nb,
                       pltpu.SemaphoreType.DMA,
                       pltpu.SemaphoreType.DMA],
        compiler_params=pltpu.CompilerParams(use_tc_tiling_on_sc=False),
    )
    def k(tbl_hbm, idx_hbm, out_hbm, idx_v, rows, gsem, ssem):
        c = lax.axis_index("c")
        s = lax.axis_index("s")
        wid = s * _NC + c
        base = wid * epw
        pltpu.sync_copy(idx_hbm.at[wid], idx_v)

        def body(g, carry):
            j0 = g * nb
            cps = [pltpu.async_copy(tbl_hbm.at[idx_v.at[j0 + b]], rows[b],
                                    gsem)
                   for b in range(nb)]
            sts = []
            for b in range(nb):
                cps[b].wait()
                sts.append(pltpu.async_copy(
                    rows[b], out_hbm.at[pl.ds(base + (j0 + b) * ch, ch)],
                    ssem))
            for st in sts:
                st.wait()
            return carry

        lax.fori_loop(0, nch // nb, body, 0)

    return k(table, idx3)


def _bmm_t(edt, es, e_real, blk):
    dout = es.shape[1]
    e_pad = es.shape[0]
    g = e_pad // blk
    gmax = -(-e_real // blk) - 1

    def body(ed_ref, es_ref, o_ref):
        est = jnp.transpose(es_ref[...])
        prod = ed_ref[...] * est[None, :, :]
        msgt = jnp.sum(prod, axis=1)
        o_ref[...] = jnp.transpose(msgt)

    return pl.pallas_call(
        body,
        grid=(g,),
        in_specs=[pl.BlockSpec((dout, dout, blk),
                               lambda i: (0, 0, jnp.minimum(i, gmax))),
                  pl.BlockSpec((blk, dout), lambda i: (i, 0))],
        out_specs=pl.BlockSpec((blk, dout), lambda i: (i, 0)),
        out_shape=jax.ShapeDtypeStruct((e_pad, dout), F32),
    )(edt, es)


def _sc_scatter(msg, idx3, zeros):
    nw, nch, ch = idx3.shape
    d = msg.shape[1]
    acc = zeros.shape[0]
    rpt = acc // _NS
    epw = nch * ch
    mesh = plsc.VectorSubcoreMesh(core_axis_name="c", subcore_axis_name="s")

    @functools.partial(
        pl.kernel,
        out_type=jax.ShapeDtypeStruct((_NC, acc, d), F32),
        mesh=mesh,
        scratch_types=[pltpu.VMEM((nch, ch), jnp.int32),
                       [pltpu.VMEM((ch, d), F32)] * 8,
                       pltpu.VMEM_SHARED((acc, d), F32),
                       pltpu.SemaphoreType.DMA,
                       pltpu.SemaphoreType.DMA],
        compiler_params=pltpu.CompilerParams(use_tc_tiling_on_sc=False),
    )
    def k(msg_hbm, idx_hbm, z_hbm, out_hbm, idx_v, bufs, acc_sh, lsem, asem):
        nb = 8
        c = lax.axis_index("c")
        s = lax.axis_index("s")
        wid = s * _NC + c
        base = wid * epw
        pltpu.sync_copy(idx_hbm.at[wid], idx_v)
        pltpu.sync_copy(z_hbm.at[pl.ds(s * rpt, rpt)],
                        acc_sh.at[pl.ds(s * rpt, rpt)])
        plsc.subcore_barrier()

        def body(g, carry):
            j0 = g * nb
            lds = [pltpu.async_copy(
                msg_hbm.at[pl.ds(base + (j0 + b) * ch, ch)], bufs[b], lsem)
                for b in range(nb)]
            adds = []
            for b in range(nb):
                lds[b].wait()
                adds.append(pltpu.async_copy(
                    bufs[b], acc_sh.at[idx_v.at[j0 + b]], asem, add=True))
            for ad in adds:
                ad.wait()
            return carry

        lax.fori_loop(0, nch // nb, body, 0)
        plsc.subcore_barrier()
        pltpu.sync_copy(acc_sh.at[pl.ds(s * rpt, rpt)],
                        out_hbm.at[c, pl.ds(s * rpt, rpt)])

    return k(msg, idx3, zeros)


def _combine(parts, bias8, n):
    dout = parts.shape[2]
    rb = 2000

    def body(p_ref, b_ref, o_ref):
        o_ref[...] = p_ref[0] + p_ref[1] + b_ref[0]

    return pl.pallas_call(
        body,
        grid=(n // rb,),
        in_specs=[pl.BlockSpec((2, rb, dout), lambda i: (0, i, 0)),
                  pl.BlockSpec((8, dout), lambda i: (0, 0))],
        out_specs=pl.BlockSpec((rb, dout), lambda i: (i, 0)),
        out_shape=jax.ShapeDtypeStruct((n, dout), F32),
    )(parts, bias8)


def kernel(input, edge_data, weight, bias, Esrc, Etgt):
    n, din = input.shape
    e = Esrc.shape[0]
    dout = weight.shape[1]

    epw = -(-e // (_NW * _CH)) * _CH
    e_pad = _NW * epw
    nch = epw // _CH
    acc = -(-(n + 1) // _NS) * _NS

    support = _support_matmul(input, weight)

    esrc3 = jnp.pad(Esrc, (0, e_pad - e)).reshape(_NW, nch, _CH)
    etgt3 = jnp.pad(Etgt, (0, e_pad - e),
                    constant_values=n).reshape(_NW, nch, _CH)

    edge_support = _sc_gather(support, esrc3)

    edt = jnp.transpose(edge_data, (1, 2, 0))
    msg = _bmm_t(edt, edge_support, e, 2048)

    zeros = jnp.zeros((acc, dout), F32)
    parts = _sc_scatter(msg, etgt3, zeros)

    bias8 = jnp.broadcast_to(bias[None, :], (8, dout))
    return _combine(parts, bias8, n)


# speedup vs baseline: 8.6635x; 1.2651x over previous
import functools

import jax
import jax.numpy as jnp
from jax import lax
from jax.experimental import pallas as pl
from jax.experimental.pallas import tpu as pltpu
from jax.experimental.pallas import tpu_sc as plsc

F32 = jnp.float32

_NC = 2
_NS = 16
_NW = _NC * _NS
_CH = 128


def _support_matmul(x, w):
    n, din = x.shape
    dout = w.shape[1]
    rb = 2000

    def body(x_ref, w_ref, o_ref):
        o_ref[...] = jnp.dot(x_ref[...], w_ref[...],
                             preferred_element_type=F32)

    return pl.pallas_call(
        body,
        grid=(n // rb,),
        in_specs=[pl.BlockSpec((rb, din), lambda i: (i, 0)),
                  pl.BlockSpec((din, dout), lambda i: (0, 0))],
        out_specs=pl.BlockSpec((rb, dout), lambda i: (i, 0)),
        out_shape=jax.ShapeDtypeStruct((n, dout), F32),
    )(x, w)


def _sc_gather(table, idx3, blk):
    nw, nch, ch = idx3.shape
    d = table.shape[1]
    epw = nch * ch
    e_pad = nw * epw
    blk4 = blk // 4
    mesh = plsc.VectorSubcoreMesh(core_axis_name="c", subcore_axis_name="s")

    nb = nch if nch <= 10 else 8
    assert nch % nb == 0

    @functools.partial(
        pl.kernel,
        out_type=jax.ShapeDtypeStruct((e_pad // 4, 128), F32),
        mesh=mesh,
        scratch_types=[pltpu.VMEM((nch, ch), jnp.int32),
                       [pltpu.VMEM((ch, d), F32)] * nb,
                       pltpu.SemaphoreType.DMA,
                       pltpu.SemaphoreType.DMA],
        compiler_params=pltpu.CompilerParams(use_tc_tiling_on_sc=False),
        name="sc_gather",
    )
    def k(tbl_hbm, idx_hbm, out_hbm, idx_v, rows, gsem, ssem):
        c = lax.axis_index("c")
        s = lax.axis_index("s")
        wid = s * _NC + c
        base = wid * epw
        pltpu.sync_copy(idx_hbm.at[wid], idx_v)

        def body(g, carry):
            j0 = g * nb
            cps = [pltpu.async_copy(tbl_hbm.at[idx_v.at[j0 + b]], rows[b],
                                    gsem)
                   for b in range(nb)]
            sts = []
            for b in range(nb):
                e0 = base + (j0 + b) * ch
                within = e0 % blk
                r0 = (e0 // blk) * blk4 + within % blk4
                q = within // blk4
                cps[b].wait()
                sts.append(pltpu.async_copy(
                    rows[b],
                    out_hbm.at[pl.ds(r0, ch), pl.ds(q * d, d)],
                    ssem))
            for st in sts:
                st.wait()
            return carry

        lax.fori_loop(0, nch // nb, body, 0)

    return k(table, idx3)


def _bmm_t(edt, es, blk, blk_off, gmax):
    dout = edt.shape[0]
    chunk = es.shape[0] * 4
    g = chunk // blk
    blk4 = blk // 4

    def body(ed_ref, es_ref, o_ref):
        es4 = es_ref[...]
        est = jnp.concatenate(
            [jnp.transpose(es4[:, q * dout:(q + 1) * dout])
             for q in range(4)], axis=1)
        prod = ed_ref[...] * est[None, :, :]
        msgt = jnp.sum(prod, axis=1)
        for q in range(4):
            o_ref[:, q * dout:(q + 1) * dout] = (
                jnp.transpose(msgt[:, q * blk4:(q + 1) * blk4]))

    return pl.pallas_call(
        body,
        grid=(g,),
        in_specs=[pl.BlockSpec((dout, dout, blk),
                               lambda i: (0, 0, jnp.minimum(blk_off + i, gmax))),
                  pl.BlockSpec((blk4, 128), lambda i: (i, 0))],
        out_specs=pl.BlockSpec((blk4, 128), lambda i: (i, 0)),
        out_shape=jax.ShapeDtypeStruct((chunk // 4, 128), F32),
    )(edt, es)


def _sc_scatter(msg, idx3, init, blk):
    nw, nch, ch = idx3.shape
    d = init.shape[2]
    acc = init.shape[1]
    rpt = acc // _NS
    epw = nch * ch
    mesh = plsc.VectorSubcoreMesh(core_axis_name="c", subcore_axis_name="s")

    @functools.partial(
        pl.kernel,
        out_type=jax.ShapeDtypeStruct((_NC, acc, d), F32),
        mesh=mesh,
        scratch_types=[pltpu.VMEM((nch, ch), jnp.int32),
                       [pltpu.VMEM((ch, d), F32)] * (nch if nch <= 10 else 8),
                       pltpu.VMEM_SHARED((acc, d), F32),
                       pltpu.SemaphoreType.DMA,
                       pltpu.SemaphoreType.DMA],
        compiler_params=pltpu.CompilerParams(use_tc_tiling_on_sc=False),
        name="sc_scatter",
    )
    def k(msg_hbm, idx_hbm, z_hbm, out_hbm, idx_v, bufs, acc_sh, lsem, asem):
        nb = nch if nch <= 10 else 8
        c = lax.axis_index("c")
        s = lax.axis_index("s")
        wid = s * _NC + c
        base = wid * epw
        pltpu.sync_copy(idx_hbm.at[wid], idx_v)
        pltpu.sync_copy(z_hbm.at[c, pl.ds(s * rpt, rpt)],
                        acc_sh.at[pl.ds(s * rpt, rpt)])
        plsc.subcore_barrier()

        def body(g, carry):
            j0 = g * nb
            lds = []
            for b in range(nb):
                e0 = base + (j0 + b) * ch
                within = e0 % blk
                r0 = (e0 // blk) * (blk // 4) + within % (blk // 4)
                q = within // (blk // 4)
                lds.append(pltpu.async_copy(
                    msg_hbm.at[pl.ds(r0, ch), pl.ds(q * d, d)],
                    bufs[b], lsem))
            adds = []
            for b in range(nb):
                lds[b].wait()
                adds.append(pltpu.async_copy(
                    bufs[b], acc_sh.at[idx_v.at[j0 + b]], asem, add=True))
            for ad in adds:
                ad.wait()
            return carry

        lax.fori_loop(0, nch // nb, body, 0)
        plsc.subcore_barrier()
        pltpu.sync_copy(acc_sh.at[pl.ds(s * rpt, rpt)],
                        out_hbm.at[c, pl.ds(s * rpt, rpt)])

    return k(msg, idx3, init)


def _combine(parts_list, bias8, n):
    dout = parts_list[0].shape[2]
    rb = 2000

    def body(*refs):
        o_ref = refs[-1]
        b_ref = refs[-2]
        tot = b_ref[0]
        for p_ref in refs[:-2]:
            tot = tot + p_ref[0] + p_ref[1]
        o_ref[...] = tot

    return pl.pallas_call(
        body,
        grid=(n // rb,),
        in_specs=[pl.BlockSpec((2, rb, dout), lambda i: (0, i, 0))
                  for _ in parts_list] +
                 [pl.BlockSpec((8, dout), lambda i: (0, 0))],
        out_specs=pl.BlockSpec((rb, dout), lambda i: (i, 0)),
        out_shape=jax.ShapeDtypeStruct((n, dout), F32),
    )(*parts_list, bias8)


def kernel(input, edge_data, weight, bias, Esrc, Etgt):
    n, din = input.shape
    e = Esrc.shape[0]
    dout = weight.shape[1]

    nchunks = 4
    blk = 4096
    cw = _NW * _CH * nchunks
    epw_tot = -(-e // cw) * _CH
    e_pad = _NW * epw_tot * nchunks
    chunk = e_pad // nchunks
    nch = chunk // (_NW * _CH)
    acc = -(-(n + 1) // _NS) * _NS
    gmax = -(-e // blk) - 1

    support = _support_matmul(input, weight)

    esrc4 = jnp.pad(Esrc, (0, e_pad - e)).reshape(nchunks, _NW, nch, _CH)
    etgt4 = jnp.pad(Etgt, (0, e_pad - e),
                    constant_values=n).reshape(nchunks, _NW, nch, _CH)

    edt = jnp.transpose(edge_data, (1, 2, 0))
    bias8 = jnp.broadcast_to(bias[None, :], (8, dout))

    parts = jnp.zeros((_NC, acc, dout), F32)
    for c in range(nchunks):
        es_c = _sc_gather(support, esrc4[c], blk)
        msg_c = _bmm_t(edt, es_c, blk, c * (chunk // blk), gmax)
        parts = _sc_scatter(msg_c, etgt4[c], parts, blk)

    return _combine([parts], bias8, n)
